# Initial kernel scaffold; baseline (speedup 1.0000x reference)
import jax
import jax.numpy as jnp
from jax.experimental import pallas as pl


def kernel(s, x):
    raise NotImplementedError("write your pallas kernel here")


# speedup vs baseline: 1.2668x; 1.2668x over previous
import jax
import jax.numpy as jnp
from jax.experimental import pallas as pl
from jax.experimental.pallas import tpu as pltpu

_GAIN = 2.0
_DECAY = 0.9
_NOISE = 0.1


def _logp_kernel(s_ref, x_ref, o_ref):
    xl = x_ref[...]
    sl = s_ref[...]
    rolled = jnp.roll(xl, 1, axis=1)
    col = jax.lax.broadcasted_iota(jnp.int32, xl.shape, 1)
    x_prev = jnp.where(col == 0, 0.0, rolled)
    bias = jax.nn.sigmoid(sl * _GAIN)
    z = (xl - _DECAY * x_prev - bias) * (1.0 / _NOISE)
    log_norm_const = jnp.log(_NOISE) + 0.5 * jnp.log(2.0 * jnp.pi)
    o_ref[...] = -0.5 * z * z - log_norm_const


@jax.jit
def kernel(s, x):
    B, T = s.shape
    BM = 128
    return pl.pallas_call(
        _logp_kernel,
        grid=(B // BM,),
        in_specs=[
            pl.BlockSpec((BM, T), lambda i: (i, 0)),
            pl.BlockSpec((BM, T), lambda i: (i, 0)),
        ],
        out_specs=pl.BlockSpec((BM, T), lambda i: (i, 0)),
        out_shape=jax.ShapeDtypeStruct((B, T), jnp.float32),
        compiler_params=pltpu.CompilerParams(
            dimension_semantics=("parallel",),
        ),
    )(s, x)


# speedup vs baseline: 1.3108x; 1.0347x over previous
import jax
import jax.numpy as jnp
from jax.experimental import pallas as pl
from jax.experimental.pallas import tpu as pltpu

_GAIN = 2.0
_DECAY = 0.9
_NOISE = 0.1


def _logp_kernel(s_ref, x_ref, o_ref):
    xl = x_ref[...]
    sl = s_ref[...]
    rolled = jnp.roll(xl, 1, axis=1)
    col = jax.lax.broadcasted_iota(jnp.int32, xl.shape, 1)
    x_prev = jnp.where(col == 0, 0.0, rolled)
    bias = jax.nn.sigmoid(sl * _GAIN)
    z = (xl - _DECAY * x_prev - bias) * (1.0 / _NOISE)
    log_norm_const = jnp.log(_NOISE) + 0.5 * jnp.log(2.0 * jnp.pi)
    o_ref[...] = -0.5 * z * z - log_norm_const


@jax.jit
def kernel(s, x):
    B, T = s.shape
    BM = 256
    return pl.pallas_call(
        _logp_kernel,
        grid=(B // BM,),
        in_specs=[
            pl.BlockSpec((BM, T), lambda i: (i, 0)),
            pl.BlockSpec((BM, T), lambda i: (i, 0)),
        ],
        out_specs=pl.BlockSpec((BM, T), lambda i: (i, 0)),
        out_shape=jax.ShapeDtypeStruct((B, T), jnp.float32),
        compiler_params=pltpu.CompilerParams(
            dimension_semantics=("parallel",),
        ),
    )(s, x)
